# baseline (speedup 1.0000x reference)
import functools

import jax
import jax.numpy as jnp
from jax import lax
from jax.experimental import pallas as pl
from jax.experimental.pallas import tpu as pltpu
from jax.experimental.pallas import tpu_sc as plsc

TOPK = 2
BT = 256
CCH = 256
SC_CORES = 2
SC_SUBCORES = 16
NW = SC_CORES * SC_SUBCORES


def _gelu_exact(x):
    return 0.5 * x * (1.0 + jax.lax.erf(x * 0.7071067811865476))


def _router_kernel(x_ref, gw_ref, logits_ref, d0_ref, d1_ref,
                   w0x_ref, w1x_ref, meta_ref, z_ref, sx_ref):
    T = x_ref.shape[0]
    E = gw_ref.shape[1]
    NT = meta_ref.shape[0]
    x = x_ref[...]
    gw = gw_ref[...]
    logits = jnp.dot(x, gw, preferred_element_type=jnp.float32)
    logits_ref[...] = logits
    col = jax.lax.broadcasted_iota(jnp.int32, (T, E), 1)
    m1 = jnp.max(logits, axis=1, keepdims=True)
    i1 = jnp.min(jnp.where(logits == m1, col, E), axis=1, keepdims=True)
    masked = jnp.where(col == i1, -jnp.inf, logits)
    m2 = jnp.max(masked, axis=1, keepdims=True)
    i2 = jnp.min(jnp.where(masked == m2, col, E), axis=1, keepdims=True)
    e2 = jnp.exp(m2 - m1)
    denom = 1.0 + e2
    w0x_ref[...] = jnp.broadcast_to(1.0 / denom, (T, 16))
    w1x_ref[...] = jnp.broadcast_to(e2 / denom, (T, 16))

    a0 = (col == i1).astype(jnp.float32)
    a1 = (col == i2).astype(jnp.float32)
    z_ref[...] = a0 + a1

    tril = (jax.lax.broadcasted_iota(jnp.int32, (CCH, CCH), 1)
            < jax.lax.broadcasted_iota(jnp.int32, (CCH, CCH), 0)
            ).astype(jnp.bfloat16)

    def chunk(c, carry):
        zc = z_ref[pl.ds(c * CCH, CCH), :]
        inc = jnp.dot(tril, zc.astype(jnp.bfloat16),
                      preferred_element_type=jnp.float32)
        sx_ref[pl.ds(c * CCH, CCH), :] = inc + carry
        return carry + jnp.sum(zc, axis=0, keepdims=True)

    counts = lax.fori_loop(0, T // CCH, chunk,
                           jnp.zeros((1, E), jnp.float32))

    sx = sx_ref[...]
    rank0 = jnp.sum(a0 * sx, axis=1, keepdims=True)
    rank1 = jnp.sum(a1 * (sx + a0), axis=1, keepdims=True)

    fBT = jnp.float32(BT)
    pc = jnp.floor((counts + (BT - 1)) * (1.0 / fBT)) * fBT
    er = jax.lax.broadcasted_iota(jnp.int32, (E, E), 0)
    ec = jax.lax.broadcasted_iota(jnp.int32, (E, E), 1)
    pc_rows = jnp.broadcast_to(pc, (E, E))
    pc_col = jnp.sum(jnp.where(er == ec, pc_rows, 0.0),
                     axis=1, keepdims=True)
    starts = jnp.sum(jnp.where(er < ec, jnp.broadcast_to(pc_col, (E, E)), 0.0),
                     axis=0, keepdims=True)
    ends = starts + pc
    total = jnp.sum(pc, axis=1, keepdims=True)

    s0 = jnp.sum(a0 * starts, axis=1, keepdims=True)
    s1 = jnp.sum(a1 * starts, axis=1, keepdims=True)
    d0_ref[...] = jnp.reshape((s0 + rank0).astype(jnp.int32), (T,))
    d1_ref[...] = jnp.reshape((s1 + rank1).astype(jnp.int32), (T,))

    g = jax.lax.broadcasted_iota(
        jnp.int32, (NT, 1), 0).astype(jnp.float32) * fBT
    pos = jnp.minimum(g, total - 1.0)
    tile_e = jnp.sum((pos >= ends).astype(jnp.int32), axis=1, keepdims=True)
    active = (g < total).astype(jnp.int32)
    meta_ref[...] = jnp.concatenate([tile_e, active], axis=1)


def _make_sc_dispatch(T, P, D):
    t_per_w = T // NW
    mesh = plsc.VectorSubcoreMesh(
        core_axis_name="c", subcore_axis_name="s",
        num_cores=SC_CORES, num_subcores=SC_SUBCORES)

    half = t_per_w // 2

    @functools.partial(
        pl.kernel, mesh=mesh,
        out_type=jax.ShapeDtypeStruct((P, D), jnp.float32),
        scratch_types=[
            pltpu.VMEM((half,), jnp.int32),
            pltpu.VMEM((half,), jnp.int32),
            pltpu.VMEM((half,), jnp.int32),
            pltpu.VMEM((half,), jnp.int32),
            pltpu.VMEM((half, D), jnp.float32),
            pltpu.VMEM((half, D), jnp.float32),
            pltpu.SemaphoreType.DMA,
            pltpu.SemaphoreType.DMA,
        ],
    )
    def k(x_hbm, d0_hbm, d1_hbm, out_hbm, i0a, i1a, i0b, i1b,
          rows_a, rows_b, sem_r, sem_w):
        wid = lax.axis_index("s") * SC_CORES + lax.axis_index("c")
        base = wid * t_per_w
        ra = pltpu.async_copy(x_hbm.at[pl.ds(base, half)], rows_a, sem_r)
        rb = pltpu.async_copy(x_hbm.at[pl.ds(base + half, half)], rows_b,
                              sem_r)
        pltpu.sync_copy(d0_hbm.at[pl.ds(base, half)], i0a)
        pltpu.sync_copy(d1_hbm.at[pl.ds(base, half)], i1a)
        pltpu.sync_copy(d0_hbm.at[pl.ds(base + half, half)], i0b)
        pltpu.sync_copy(d1_hbm.at[pl.ds(base + half, half)], i1b)
        ra.wait()
        s0 = pltpu.async_copy(rows_a, out_hbm.at[i0a], sem_w)
        s1 = pltpu.async_copy(rows_a, out_hbm.at[i1a], sem_w)
        rb.wait()
        s2 = pltpu.async_copy(rows_b, out_hbm.at[i0b], sem_w)
        s3 = pltpu.async_copy(rows_b, out_hbm.at[i1b], sem_w)
        s0.wait()
        s1.wait()
        s2.wait()
        s3.wait()

    return k


def _make_sc_combine(T, P, D, chunk):
    t_per_w = T // NW
    mesh = plsc.VectorSubcoreMesh(
        core_axis_name="c", subcore_axis_name="s",
        num_cores=SC_CORES, num_subcores=SC_SUBCORES)

    @functools.partial(
        pl.kernel, mesh=mesh,
        out_type=jax.ShapeDtypeStruct((T, D), jnp.float32),
        scratch_types=[
            pltpu.VMEM((chunk,), jnp.int32),
            pltpu.VMEM((chunk,), jnp.int32),
            pltpu.VMEM((chunk,), jnp.int32),
            pltpu.VMEM((chunk,), jnp.int32),
            pltpu.VMEM((chunk, 16), jnp.float32),
            pltpu.VMEM((chunk, 16), jnp.float32),
            pltpu.VMEM((chunk, 16), jnp.float32),
            pltpu.VMEM((chunk, 16), jnp.float32),
            pltpu.VMEM((chunk, D), jnp.float32),
            pltpu.VMEM((chunk, D), jnp.float32),
            pltpu.VMEM((chunk, D), jnp.float32),
            pltpu.VMEM((chunk, D), jnp.float32),
            pltpu.SemaphoreType.DMA,
            pltpu.SemaphoreType.DMA,
            pltpu.SemaphoreType.DMA,
        ],
    )
    def k(y_hbm, d0_hbm, d1_hbm, w0_hbm, w1_hbm, out_hbm,
          i0a, i1a, i0b, i1b, w0a, w1a, w0b, w1b,
          b0a, b1a, b0b, b1b, sem_a, sem_b, sem_s):
        wid = lax.axis_index("s") * SC_CORES + lax.axis_index("c")
        base = wid * t_per_w
        nlane = D // 16

        def add_rows(b0, b1, w0_v, w1_v):
            def row(r, _):
                w0 = w0_v[r, :]
                w1 = w1_v[r, :]
                for cc in range(nlane):
                    sl = pl.ds(cc * 16, 16)
                    b0[r, sl] = b0[r, sl] * w0 + b1[r, sl] * w1
                return 0

            lax.fori_loop(0, chunk, row, 0)

        offa = base
        offb = base + chunk
        pltpu.sync_copy(d0_hbm.at[pl.ds(offa, chunk)], i0a)
        pltpu.sync_copy(d1_hbm.at[pl.ds(offa, chunk)], i1a)
        g0a = pltpu.async_copy(y_hbm.at[i0a], b0a, sem_a)
        g1a = pltpu.async_copy(y_hbm.at[i1a], b1a, sem_a)
        pltpu.sync_copy(d0_hbm.at[pl.ds(offb, chunk)], i0b)
        pltpu.sync_copy(d1_hbm.at[pl.ds(offb, chunk)], i1b)
        g0b = pltpu.async_copy(y_hbm.at[i0b], b0b, sem_b)
        g1b = pltpu.async_copy(y_hbm.at[i1b], b1b, sem_b)
        pltpu.sync_copy(w0_hbm.at[pl.ds(offa, chunk)], w0a)
        pltpu.sync_copy(w1_hbm.at[pl.ds(offa, chunk)], w1a)
        pltpu.sync_copy(w0_hbm.at[pl.ds(offb, chunk)], w0b)
        pltpu.sync_copy(w1_hbm.at[pl.ds(offb, chunk)], w1b)
        g0a.wait()
        g1a.wait()
        add_rows(b0a, b1a, w0a, w1a)
        sa = pltpu.async_copy(b0a, out_hbm.at[pl.ds(offa, chunk)], sem_s)
        g0b.wait()
        g1b.wait()
        add_rows(b0b, b1b, w0b, w1b)
        sb = pltpu.async_copy(b0b, out_hbm.at[pl.ds(offb, chunk)], sem_s)
        sa.wait()
        sb.wait()

    return k


def _ffn_kernel(meta_ref, xg_ref, w1_ref, w2_ref, y_ref):
    g = pl.program_id(0)
    ii = pl.program_id(1)

    @pl.when(meta_ref[g, 1] == 1)
    def _():
        xb = xg_ref[...].astype(jnp.bfloat16)
        h = jnp.dot(xb, w1_ref[0].astype(jnp.bfloat16),
                    preferred_element_type=jnp.float32)
        h = _gelu_exact(h)
        o = jnp.dot(h.astype(jnp.bfloat16),
                    w2_ref[0].astype(jnp.bfloat16),
                    preferred_element_type=jnp.float32)

        @pl.when(ii == 0)
        def _init():
            y_ref[...] = o

        @pl.when(ii != 0)
        def _acc():
            y_ref[...] += o


@jax.jit
def kernel(hidden_states, gate_w, w1, w2):
    B, S, H = hidden_states.shape
    E = gate_w.shape[1]
    I = w1.shape[2]
    T = B * S
    A = T * TOPK
    P = A + E * BT
    NT = P // BT
    flat = hidden_states.reshape(T, H)

    logits, d0, d1, pw0, pw1, meta = pl.pallas_call(
        _router_kernel,
        out_shape=(
            jax.ShapeDtypeStruct((T, E), jnp.float32),
            jax.ShapeDtypeStruct((T,), jnp.int32),
            jax.ShapeDtypeStruct((T,), jnp.int32),
            jax.ShapeDtypeStruct((T, 16), jnp.float32),
            jax.ShapeDtypeStruct((T, 16), jnp.float32),
            jax.ShapeDtypeStruct((NT, 2), jnp.int32),
        ),
        scratch_shapes=[
            pltpu.VMEM((T, E), jnp.float32),
            pltpu.VMEM((T, E), jnp.float32),
        ],
    )(flat, gate_w)

    xg = _make_sc_dispatch(T, P, H)(flat, d0, d1)

    NI = 2
    grid_spec = pltpu.PrefetchScalarGridSpec(
        num_scalar_prefetch=1,
        grid=(NT, NI),
        in_specs=[
            pl.BlockSpec((BT, H), lambda g, ii, m: (g, 0)),
            pl.BlockSpec((1, H, I // NI), lambda g, ii, m: (m[g, 0], 0, ii)),
            pl.BlockSpec((1, I // NI, H), lambda g, ii, m: (m[g, 0], ii, 0)),
        ],
        out_specs=pl.BlockSpec((BT, H), lambda g, ii, m: (g, 0)),
    )
    y = pl.pallas_call(
        _ffn_kernel,
        grid_spec=grid_spec,
        out_shape=jax.ShapeDtypeStruct((P, H), jnp.float32),
    )(meta, xg, w1, w2)

    out = _make_sc_combine(T, P, H, 32)(y, d0, d1, pw0, pw1)

    return out.reshape(B, S, H), logits.reshape(B, S, E)


# speedup vs baseline: 1.4369x; 1.4369x over previous
import functools

import jax
import jax.numpy as jnp
from jax import lax
from jax.experimental import pallas as pl
from jax.experimental.pallas import tpu as pltpu
from jax.experimental.pallas import tpu_sc as plsc

TOPK = 2
BT = 512
CCH = 256
SC_CORES = 2
SC_SUBCORES = 16
NW = SC_CORES * SC_SUBCORES


def _gelu_exact(x):
    return 0.5 * x * (1.0 + jax.lax.erf(x * 0.7071067811865476))


def _router_kernel(x_ref, gw_ref, logits_ref, d0_ref, d1_ref,
                   w0x_ref, w1x_ref, meta_ref, z_ref, sx_ref):
    T = x_ref.shape[0]
    E = gw_ref.shape[1]
    NT = meta_ref.shape[0]
    x = x_ref[...]
    gw = gw_ref[...]
    logits = jnp.dot(x, gw, preferred_element_type=jnp.float32)
    logits_ref[...] = logits
    col = jax.lax.broadcasted_iota(jnp.int32, (T, E), 1)
    m1 = jnp.max(logits, axis=1, keepdims=True)
    i1 = jnp.min(jnp.where(logits == m1, col, E), axis=1, keepdims=True)
    masked = jnp.where(col == i1, -jnp.inf, logits)
    m2 = jnp.max(masked, axis=1, keepdims=True)
    i2 = jnp.min(jnp.where(masked == m2, col, E), axis=1, keepdims=True)
    e2 = jnp.exp(m2 - m1)
    denom = 1.0 + e2
    w0x_ref[...] = jnp.broadcast_to(1.0 / denom, (T, 16))
    w1x_ref[...] = jnp.broadcast_to(e2 / denom, (T, 16))

    a0 = (col == i1).astype(jnp.float32)
    a1 = (col == i2).astype(jnp.float32)
    z_ref[...] = a0 + a1

    tril = (jax.lax.broadcasted_iota(jnp.int32, (CCH, CCH), 1)
            < jax.lax.broadcasted_iota(jnp.int32, (CCH, CCH), 0)
            ).astype(jnp.bfloat16)

    def chunk(c, carry):
        zc = z_ref[pl.ds(c * CCH, CCH), :]
        inc = jnp.dot(tril, zc.astype(jnp.bfloat16),
                      preferred_element_type=jnp.float32)
        sx_ref[pl.ds(c * CCH, CCH), :] = inc + carry
        return carry + jnp.sum(zc, axis=0, keepdims=True)

    counts = lax.fori_loop(0, T // CCH, chunk,
                           jnp.zeros((1, E), jnp.float32))

    sx = sx_ref[...]
    rank0 = jnp.sum(a0 * sx, axis=1, keepdims=True)
    rank1 = jnp.sum(a1 * (sx + a0), axis=1, keepdims=True)

    fBT = jnp.float32(BT)
    pc = jnp.floor((counts + (BT - 1)) * (1.0 / fBT)) * fBT
    er = jax.lax.broadcasted_iota(jnp.int32, (E, E), 0)
    ec = jax.lax.broadcasted_iota(jnp.int32, (E, E), 1)
    pc_rows = jnp.broadcast_to(pc, (E, E))
    pc_col = jnp.sum(jnp.where(er == ec, pc_rows, 0.0),
                     axis=1, keepdims=True)
    starts = jnp.sum(jnp.where(er < ec, jnp.broadcast_to(pc_col, (E, E)), 0.0),
                     axis=0, keepdims=True)
    ends = starts + pc
    total = jnp.sum(pc, axis=1, keepdims=True)

    s0 = jnp.sum(a0 * starts, axis=1, keepdims=True)
    s1 = jnp.sum(a1 * starts, axis=1, keepdims=True)
    d0_ref[...] = jnp.reshape((s0 + rank0).astype(jnp.int32), (T,))
    d1_ref[...] = jnp.reshape((s1 + rank1).astype(jnp.int32), (T,))

    g = jax.lax.broadcasted_iota(
        jnp.int32, (NT, 1), 0).astype(jnp.float32) * fBT
    pos = jnp.minimum(g, total - 1.0)
    tile_e = jnp.sum((pos >= ends).astype(jnp.int32), axis=1, keepdims=True)
    active = (g < total).astype(jnp.int32)
    meta_ref[...] = jnp.concatenate([tile_e, active], axis=1)


def _make_sc_dispatch(T, P, D):
    t_per_w = T // NW
    mesh = plsc.VectorSubcoreMesh(
        core_axis_name="c", subcore_axis_name="s",
        num_cores=SC_CORES, num_subcores=SC_SUBCORES)

    half = t_per_w // 2

    @functools.partial(
        pl.kernel, mesh=mesh,
        out_type=jax.ShapeDtypeStruct((P, D), jnp.float32),
        scratch_types=[
            pltpu.VMEM((half,), jnp.int32),
            pltpu.VMEM((half,), jnp.int32),
            pltpu.VMEM((half,), jnp.int32),
            pltpu.VMEM((half,), jnp.int32),
            pltpu.VMEM((half, D), jnp.float32),
            pltpu.VMEM((half, D), jnp.float32),
            pltpu.SemaphoreType.DMA,
            pltpu.SemaphoreType.DMA,
        ],
    )
    def k(x_hbm, d0_hbm, d1_hbm, out_hbm, i0a, i1a, i0b, i1b,
          rows_a, rows_b, sem_r, sem_w):
        wid = lax.axis_index("s") * SC_CORES + lax.axis_index("c")
        base = wid * t_per_w
        ra = pltpu.async_copy(x_hbm.at[pl.ds(base, half)], rows_a, sem_r)
        rb = pltpu.async_copy(x_hbm.at[pl.ds(base + half, half)], rows_b,
                              sem_r)
        pltpu.sync_copy(d0_hbm.at[pl.ds(base, half)], i0a)
        pltpu.sync_copy(d1_hbm.at[pl.ds(base, half)], i1a)
        pltpu.sync_copy(d0_hbm.at[pl.ds(base + half, half)], i0b)
        pltpu.sync_copy(d1_hbm.at[pl.ds(base + half, half)], i1b)
        ra.wait()
        s0 = pltpu.async_copy(rows_a, out_hbm.at[i0a], sem_w)
        s1 = pltpu.async_copy(rows_a, out_hbm.at[i1a], sem_w)
        rb.wait()
        s2 = pltpu.async_copy(rows_b, out_hbm.at[i0b], sem_w)
        s3 = pltpu.async_copy(rows_b, out_hbm.at[i1b], sem_w)
        s0.wait()
        s1.wait()
        s2.wait()
        s3.wait()

    return k


def _make_sc_combine(T, P, D, chunk):
    t_per_w = T // NW
    mesh = plsc.VectorSubcoreMesh(
        core_axis_name="c", subcore_axis_name="s",
        num_cores=SC_CORES, num_subcores=SC_SUBCORES)

    @functools.partial(
        pl.kernel, mesh=mesh,
        out_type=jax.ShapeDtypeStruct((T, D), jnp.float32),
        scratch_types=[
            pltpu.VMEM((chunk,), jnp.int32),
            pltpu.VMEM((chunk,), jnp.int32),
            pltpu.VMEM((chunk,), jnp.int32),
            pltpu.VMEM((chunk,), jnp.int32),
            pltpu.VMEM((chunk, 16), jnp.float32),
            pltpu.VMEM((chunk, 16), jnp.float32),
            pltpu.VMEM((chunk, 16), jnp.float32),
            pltpu.VMEM((chunk, 16), jnp.float32),
            pltpu.VMEM((chunk, D), jnp.float32),
            pltpu.VMEM((chunk, D), jnp.float32),
            pltpu.VMEM((chunk, D), jnp.float32),
            pltpu.VMEM((chunk, D), jnp.float32),
            pltpu.SemaphoreType.DMA,
            pltpu.SemaphoreType.DMA,
            pltpu.SemaphoreType.DMA,
        ],
    )
    def k(y_hbm, d0_hbm, d1_hbm, w0_hbm, w1_hbm, out_hbm,
          i0a, i1a, i0b, i1b, w0a, w1a, w0b, w1b,
          b0a, b1a, b0b, b1b, sem_a, sem_b, sem_s):
        wid = lax.axis_index("s") * SC_CORES + lax.axis_index("c")
        base = wid * t_per_w
        nlane = D // 16

        def add_rows(b0, b1, w0_v, w1_v):
            def row(r, _):
                w0 = w0_v[r, :]
                w1 = w1_v[r, :]
                for cc in range(nlane):
                    sl = pl.ds(cc * 16, 16)
                    b0[r, sl] = b0[r, sl] * w0 + b1[r, sl] * w1
                return 0

            lax.fori_loop(0, chunk, row, 0)

        offa = base
        offb = base + chunk
        pltpu.sync_copy(d0_hbm.at[pl.ds(offa, chunk)], i0a)
        pltpu.sync_copy(d1_hbm.at[pl.ds(offa, chunk)], i1a)
        g0a = pltpu.async_copy(y_hbm.at[i0a], b0a, sem_a)
        g1a = pltpu.async_copy(y_hbm.at[i1a], b1a, sem_a)
        pltpu.sync_copy(d0_hbm.at[pl.ds(offb, chunk)], i0b)
        pltpu.sync_copy(d1_hbm.at[pl.ds(offb, chunk)], i1b)
        g0b = pltpu.async_copy(y_hbm.at[i0b], b0b, sem_b)
        g1b = pltpu.async_copy(y_hbm.at[i1b], b1b, sem_b)
        pltpu.sync_copy(w0_hbm.at[pl.ds(offa, chunk)], w0a)
        pltpu.sync_copy(w1_hbm.at[pl.ds(offa, chunk)], w1a)
        pltpu.sync_copy(w0_hbm.at[pl.ds(offb, chunk)], w0b)
        pltpu.sync_copy(w1_hbm.at[pl.ds(offb, chunk)], w1b)
        g0a.wait()
        g1a.wait()
        add_rows(b0a, b1a, w0a, w1a)
        sa = pltpu.async_copy(b0a, out_hbm.at[pl.ds(offa, chunk)], sem_s)
        g0b.wait()
        g1b.wait()
        add_rows(b0b, b1b, w0b, w1b)
        sb = pltpu.async_copy(b0b, out_hbm.at[pl.ds(offb, chunk)], sem_s)
        sa.wait()
        sb.wait()

    return k


def _ffn_kernel(meta_ref, xg_ref, w1_ref, w2_ref, y_ref):
    g = pl.program_id(0)

    @pl.when(meta_ref[g, 1] == 1)
    def _():
        xb = xg_ref[...].astype(jnp.bfloat16)
        h = jnp.dot(xb, w1_ref[0].astype(jnp.bfloat16),
                    preferred_element_type=jnp.float32)
        h = _gelu_exact(h)
        y_ref[...] = jnp.dot(h.astype(jnp.bfloat16),
                             w2_ref[0].astype(jnp.bfloat16),
                             preferred_element_type=jnp.float32)


@jax.jit
def kernel(hidden_states, gate_w, w1, w2):
    B, S, H = hidden_states.shape
    E = gate_w.shape[1]
    I = w1.shape[2]
    T = B * S
    A = T * TOPK
    P = A + E * BT
    NT = P // BT
    flat = hidden_states.reshape(T, H)

    logits, d0, d1, pw0, pw1, meta = pl.pallas_call(
        _router_kernel,
        out_shape=(
            jax.ShapeDtypeStruct((T, E), jnp.float32),
            jax.ShapeDtypeStruct((T,), jnp.int32),
            jax.ShapeDtypeStruct((T,), jnp.int32),
            jax.ShapeDtypeStruct((T, 16), jnp.float32),
            jax.ShapeDtypeStruct((T, 16), jnp.float32),
            jax.ShapeDtypeStruct((NT, 2), jnp.int32),
        ),
        scratch_shapes=[
            pltpu.VMEM((T, E), jnp.float32),
            pltpu.VMEM((T, E), jnp.float32),
        ],
    )(flat, gate_w)

    xg = _make_sc_dispatch(T, P, H)(flat, d0, d1)

    grid_spec = pltpu.PrefetchScalarGridSpec(
        num_scalar_prefetch=1,
        grid=(NT,),
        in_specs=[
            pl.BlockSpec((BT, H), lambda g, m: (g, 0)),
            pl.BlockSpec((1, H, I), lambda g, m: (m[g, 0], 0, 0)),
            pl.BlockSpec((1, I, H), lambda g, m: (m[g, 0], 0, 0)),
        ],
        out_specs=pl.BlockSpec((BT, H), lambda g, m: (g, 0)),
    )
    y = pl.pallas_call(
        _ffn_kernel,
        grid_spec=grid_spec,
        out_shape=jax.ShapeDtypeStruct((P, H), jnp.float32),
    )(meta, xg, w1, w2)

    out = _make_sc_combine(T, P, H, 32)(y, d0, d1, pw0, pw1)

    return out.reshape(B, S, H), logits.reshape(B, S, E)
